# baseline (speedup 1.0000x reference)
import functools

import jax
import jax.numpy as jnp
import numpy as np
from jax import lax
from jax.experimental import pallas as pl
from jax.experimental.pallas import tpu as pltpu
from jax.experimental.pallas import tpu_sc as plsc

B = 4096
L = 200
D = 64
NC = 2
NS = 16
NW = NC * NS
BW = B // NW
NLG = L // 8


def _pe_splat():
    position = np.arange(0, L, dtype=np.float32)[:, None]
    div_term = np.exp(np.arange(0, D, 2, dtype=np.float32) * (-np.log(10000.0) / D))
    pe = np.zeros((L, D), dtype=np.float32)
    pe[:, 0::2] = np.sin(position * div_term)
    pe[:, 1::2] = np.cos(position * div_term)
    splat = np.repeat(pe.reshape(L, D, 1), 16, axis=2)
    return jnp.asarray(splat.reshape(NLG, 8, 8, 128))


_MESH = plsc.VectorSubcoreMesh(core_axis_name="c", subcore_axis_name="s")


@functools.partial(
    pl.kernel,
    out_type=jax.ShapeDtypeStruct((L, D, B), jnp.float32),
    mesh=_MESH,
    scratch_types=[
        pltpu.VMEM((8, BW), jnp.int32),
        pltpu.VMEM((2, BW), jnp.int32),
        pltpu.VMEM((2, BW), jnp.int32),
        pltpu.VMEM((8, 8, 128), jnp.float32),
        pltpu.VMEM((2, BW, 2 * D), jnp.float32),
        pltpu.VMEM((2, D, BW), jnp.float32),
        pltpu.SemaphoreType.DMA,
        pltpu.SemaphoreType.DMA,
        pltpu.SemaphoreType.DMA,
        pltpu.SemaphoreType.DMA,
    ],
    compiler_params=pltpu.CompilerParams(use_tc_tiling_on_sc=True, needs_layout_passes=False),
)
def _embed(idx3, pesp, table2, out3,
           idx_v, jv_v, hv_v, pes_v, rows_v, trans_v,
           semg0, semg1, semo0, semo1):
    w = lax.axis_index("s") * NC + lax.axis_index("c")
    bcol = w * BW
    semg = (semg0, semg1)
    semo = (semo0, semo1)
    iota = jnp.arange(16, dtype=jnp.int32)

    def load_blocks(lg):
        pltpu.sync_copy(idx3.at[lg, :, pl.ds(bcol, BW)], idx_v)
        pltpu.sync_copy(pesp.at[lg], pes_v)

    def prep(r, nb):
        for k in range(BW // 16):
            iv = idx_v[r, pl.ds(16 * k, 16)]
            jv_v[nb, pl.ds(16 * k, 16)] = lax.shift_right_logical(iv, 1)
            hv_v[nb, pl.ds(16 * k, 16)] = lax.shift_left(iv & 1, 6)

    def fire_gather(nb):
        pltpu.async_copy(table2.at[jv_v.at[nb]], rows_v.at[nb], semg[nb])

    def wait_gather(nb):
        pltpu.make_async_copy(
            table2.at[jv_v.at[nb]], rows_v.at[nb], semg[nb]
        ).wait()

    def fire_store(l, b):
        pltpu.async_copy(trans_v.at[b], out3.at[l, :, pl.ds(bcol, BW)], semo[b])

    def wait_store(b):
        pltpu.make_async_copy(
            trans_v.at[b], out3.at[0, :, pl.ds(bcol, BW)], semo[b]
        ).wait()

    def compute(r, b):
        hvs = [hv_v[b, pl.ds(16 * g, 16)] for g in range(BW // 16)]
        rowvs = [iota + 16 * g for g in range(BW // 16)]

        def dbody(d, c):
            pvec = pes_v[r, d // 8, pl.ds((d % 8) * 16, 16)]
            for g in range(BW // 16):
                vals = plsc.load_gather(rows_v.at[b], [rowvs[g], hvs[g] + d])
                trans_v[b, d, pl.ds(16 * g, 16)] = vals + pvec
            return c

        lax.fori_loop(0, D, dbody, 0)

    load_blocks(0)
    prep(0, 0)
    fire_gather(0)

    def lg_body(lg, carry):
        for r in range(8):
            l = 8 * lg + r
            b = r % 2
            nb = 1 - b
            if r < 7:
                prep(r + 1, nb)
                fire_gather(nb)
            wait_gather(b)
            if r < 2:
                @pl.when(lg >= 1)
                def _():
                    wait_store(b)
            else:
                wait_store(b)
            compute(r, b)
            fire_store(l, b)
            if r == 7:
                @pl.when(lg < NLG - 1)
                def _():
                    load_blocks(lg + 1)
                    prep(0, nb)
                    fire_gather(nb)
        return carry

    lax.fori_loop(0, NLG, lg_body, 0)
    wait_store(0)
    wait_store(1)


def kernel(indices, table):
    idx3 = indices.T.reshape(NLG, 8, B).astype(jnp.int32)
    table2 = table.reshape(table.shape[0] // 2, 2 * D)
    out3 = _embed(idx3, _pe_splat(), table2)
    return jnp.transpose(out3, (2, 0, 1))


# speedup vs baseline: 1.8373x; 1.8373x over previous
import functools

import jax
import jax.numpy as jnp
import numpy as np
from jax import lax
from jax.experimental import pallas as pl
from jax.experimental.pallas import tpu as pltpu
from jax.experimental.pallas import tpu_sc as plsc

B = 4096
L = 200
D = 64
NC = 2
NS = 16
NW = NC * NS
BW = B // NW
NLG = L // 8


def _pe_splat():
    position = np.arange(0, L, dtype=np.float32)[:, None]
    div_term = np.exp(np.arange(0, D, 2, dtype=np.float32) * (-np.log(10000.0) / D))
    pe = np.zeros((L, D), dtype=np.float32)
    pe[:, 0::2] = np.sin(position * div_term)
    pe[:, 1::2] = np.cos(position * div_term)
    splat = np.repeat(pe.reshape(L, D, 1), 16, axis=2)
    return jnp.asarray(splat.reshape(NLG, 8, 8, 128))


_MESH = plsc.VectorSubcoreMesh(core_axis_name="c", subcore_axis_name="s")


@functools.partial(
    pl.kernel,
    out_type=jax.ShapeDtypeStruct((L, D, B), jnp.float32),
    mesh=_MESH,
    scratch_types=[
        pltpu.VMEM((8, BW), jnp.int32),
        pltpu.VMEM((2, BW), jnp.int32),
        pltpu.VMEM((2, BW), jnp.int32),
        pltpu.VMEM((8, 8, 128), jnp.float32),
        pltpu.VMEM((2, BW, 2 * D), jnp.float32),
        pltpu.VMEM((2, D, BW), jnp.float32),
        pltpu.SemaphoreType.DMA,
        pltpu.SemaphoreType.DMA,
        pltpu.SemaphoreType.DMA,
        pltpu.SemaphoreType.DMA,
    ],
    compiler_params=pltpu.CompilerParams(use_tc_tiling_on_sc=True, needs_layout_passes=False),
)
def _embed(idx3, pesp, table2, out3,
           idx_v, jv_v, hv_v, pes_v, rows_v, trans_v,
           semg0, semg1, semo0, semo1):
    w = lax.axis_index("s") * NC + lax.axis_index("c")
    bcol = w * BW
    semg = (semg0, semg1)
    semo = (semo0, semo1)
    iota = jnp.arange(16, dtype=jnp.int32)

    def load_blocks(lg):
        pltpu.sync_copy(idx3.at[lg, :, pl.ds(bcol, BW)], idx_v)
        pltpu.sync_copy(pesp.at[lg], pes_v)

    def prep(r, nb):
        for k in range(BW // 16):
            iv = idx_v[r, pl.ds(16 * k, 16)]
            jv_v[nb, pl.ds(16 * k, 16)] = lax.shift_right_logical(iv, 1)
            hv_v[nb, pl.ds(16 * k, 16)] = lax.shift_left(iv & 1, 6)

    def fire_gather(nb):
        pltpu.async_copy(table2.at[jv_v.at[nb]], rows_v.at[nb], semg[nb])

    def wait_gather(nb):
        pltpu.make_async_copy(
            table2.at[jv_v.at[nb]], rows_v.at[nb], semg[nb]
        ).wait()

    def fire_store(l, b):
        pltpu.async_copy(trans_v.at[b], out3.at[l, :, pl.ds(bcol, BW)], semo[b])

    def wait_store(b):
        pltpu.make_async_copy(
            trans_v.at[b], out3.at[0, :, pl.ds(bcol, BW)], semo[b]
        ).wait()

    def compute(r, b):
        hvs = [hv_v[b, pl.ds(16 * g, 16)] for g in range(BW // 16)]
        rowvs = [iota + 16 * g for g in range(BW // 16)]

        @plsc.parallel_loop(0, D, step=1, unroll=4)
        def dbody(d):
            pvec = pes_v[r, d // 8, pl.ds((d % 8) * 16, 16)]
            for g in range(BW // 16):
                vals = plsc.load_gather(rows_v.at[b], [rowvs[g], hvs[g] + d])
                trans_v[b, d, pl.ds(16 * g, 16)] = vals + pvec

    load_blocks(0)
    prep(0, 0)
    fire_gather(0)

    def lg_body(lg, carry):
        for r in range(8):
            l = 8 * lg + r
            b = r % 2
            nb = 1 - b
            if r < 7:
                prep(r + 1, nb)
                fire_gather(nb)
            wait_gather(b)
            if r < 2:
                @pl.when(lg >= 1)
                def _():
                    wait_store(b)
            else:
                wait_store(b)
            compute(r, b)
            fire_store(l, b)
            if r == 7:
                @pl.when(lg < NLG - 1)
                def _():
                    load_blocks(lg + 1)
                    prep(0, nb)
                    fire_gather(nb)
        return carry

    lax.fori_loop(0, NLG, lg_body, 0)
    wait_store(0)
    wait_store(1)


def kernel(indices, table):
    idx3 = indices.T.reshape(NLG, 8, B).astype(jnp.int32)
    table2 = table.reshape(table.shape[0] // 2, 2 * D)
    out3 = _embed(idx3, _pe_splat(), table2)
    return jnp.transpose(out3, (2, 0, 1))


# speedup vs baseline: 1.9709x; 1.0727x over previous
import functools

import jax
import jax.numpy as jnp
import numpy as np
from jax import lax
from jax.experimental import pallas as pl
from jax.experimental.pallas import tpu as pltpu
from jax.experimental.pallas import tpu_sc as plsc

B = 4096
L = 200
D = 64
NC = 2
NS = 16
NW = NC * NS
BW = B // NW
NLG = L // 8
NG = BW // 16


def _pos_encoding():
    position = np.arange(0, L, dtype=np.float32)[:, None]
    div_term = np.exp(np.arange(0, D, 2, dtype=np.float32) * (-np.log(10000.0) / D))
    pe = np.zeros((L, D), dtype=np.float32)
    pe[:, 0::2] = np.sin(position * div_term)
    pe[:, 1::2] = np.cos(position * div_term)
    return jnp.asarray(pe.reshape(L // 2, 2 * D))


_MESH = plsc.VectorSubcoreMesh(core_axis_name="c", subcore_axis_name="s")


@functools.partial(
    pl.kernel,
    out_type=jax.ShapeDtypeStruct((L, D, B), jnp.float32),
    mesh=_MESH,
    scratch_types=[
        pltpu.VMEM((NLG, 8, BW), jnp.int32),
        pltpu.VMEM((4, BW), jnp.int32),
        pltpu.VMEM((4, BW), jnp.int32),
        pltpu.VMEM((L // 2, 2 * D), jnp.float32),
        pltpu.VMEM((4, BW, 2 * D), jnp.float32),
        pltpu.VMEM((2, D, BW), jnp.float32),
        pltpu.SemaphoreType.DMA,
        pltpu.SemaphoreType.DMA,
        pltpu.SemaphoreType.DMA,
        pltpu.SemaphoreType.DMA,
        pltpu.SemaphoreType.DMA,
        pltpu.SemaphoreType.DMA,
    ],
    compiler_params=pltpu.CompilerParams(
        use_tc_tiling_on_sc=True, needs_layout_passes=False
    ),
)
def _embed(idx3, pe2, table2, out3,
           idx_all, jv_v, hv_v, pe_v, rows_v, trans_v,
           semg0, semg1, semg2, semg3, semo0, semo1):
    w = lax.axis_index("s") * NC + lax.axis_index("c")
    bcol = w * BW
    semg = (semg0, semg1, semg2, semg3)
    semo = (semo0, semo1)
    iota = jnp.arange(16, dtype=jnp.int32)

    pltpu.sync_copy(idx3.at[:, :, pl.ds(bcol, BW)], idx_all)
    pltpu.sync_copy(pe2, pe_v)

    def prep(lgx, rx, nb):
        for k in range(NG):
            iv = idx_all[lgx, rx, pl.ds(16 * k, 16)]
            jv_v[nb, pl.ds(16 * k, 16)] = lax.shift_right_logical(iv, 1)
            hv_v[nb, pl.ds(16 * k, 16)] = lax.shift_left(iv & 1, 6)

    def fire_gather(nb):
        pltpu.async_copy(table2.at[jv_v.at[nb]], rows_v.at[nb], semg[nb])

    def wait_gather(nb):
        pltpu.make_async_copy(
            table2.at[jv_v.at[nb]], rows_v.at[nb], semg[nb]
        ).wait()

    def fire_store(l, b):
        pltpu.async_copy(trans_v.at[b], out3.at[l, :, pl.ds(bcol, BW)], semo[b])

    def wait_store(b):
        pltpu.make_async_copy(
            trans_v.at[b], out3.at[0, :, pl.ds(bcol, BW)], semo[b]
        ).wait()

    def compute(l, r, b, gb):
        lvec = jnp.full((16,), 0, jnp.int32) + l // 2
        cbase = jnp.full((16,), (r % 2) * D, jnp.int32)
        hvs = [hv_v[gb, pl.ds(16 * g, 16)] for g in range(NG)]
        rowvs = [iota + 16 * g for g in range(NG)]

        @plsc.parallel_loop(0, D, step=1, unroll=4)
        def dbody(d):
            pvec = plsc.load_gather(pe_v, [lvec, cbase + d])
            for g in range(NG):
                vals = plsc.load_gather(rows_v.at[gb], [rowvs[g], hvs[g] + d])
                trans_v[b, d, pl.ds(16 * g, 16)] = vals + pvec

    prep(0, 0, 0)
    fire_gather(0)
    prep(0, 1, 1)
    fire_gather(1)

    def lg_body(lg, carry):
        for r in range(8):
            l = 8 * lg + r
            b = r % 2
            gb = r % 4
            pb = (r + 2) % 4
            if r < 6:
                prep(lg, r + 2, pb)
                fire_gather(pb)
            else:
                @pl.when(lg < NLG - 1)
                def _():
                    prep(lg + 1, r - 6, pb)
                    fire_gather(pb)
            wait_gather(gb)
            if r < 2:
                @pl.when(lg >= 1)
                def _():
                    wait_store(b)
            else:
                wait_store(b)
            compute(l, r, b, gb)
            fire_store(l, b)
        return carry

    lax.fori_loop(0, NLG, lg_body, 0)
    wait_store(0)
    wait_store(1)


def kernel(indices, table):
    idx3 = indices.T.reshape(NLG, 8, B).astype(jnp.int32)
    table2 = table.reshape(table.shape[0] // 2, 2 * D)
    out3 = _embed(idx3, _pos_encoding(), table2)
    return jnp.transpose(out3, (2, 0, 1))
